# baseline (speedup 1.0000x reference)
import functools

import jax
import jax.numpy as jnp
from jax import lax
from jax.experimental import pallas as pl
from jax.experimental.pallas import tpu as pltpu
from jax.experimental.pallas import tpu_sc as plsc

VOCAB = 1000
NE = 32
T = 8
BATCH = 4096
BF = BATCH * T

NC = 2
NS = 16
NW = NC * NS
BPW = BF // NW
CH = 64
NCHUNK = BPW // CH


def _tables_body(idx_ref, tok_ref, pos_ref, w_ref, b_ref, comb_ref, fidx_ref):
    t = pl.program_id(0)

    @pl.when(t == 0)
    def _():
        pos_of_lane = lax.broadcasted_iota(jnp.int32, (BF // 128, 128), 1) % T
        fidx_ref[...] = idx_ref[...] + pos_of_lane * VOCAB

    x = tok_ref[...] + pos_ref[0]
    comb_ref[...] = (
        lax.dot_general(
            x,
            w_ref[...],
            (((1,), (1,)), ((), ())),
            precision=lax.Precision.HIGHEST,
            preferred_element_type=jnp.float32,
        )
        + b_ref[...]
    )


_tables = pl.pallas_call(
    _tables_body,
    grid=(T,),
    in_specs=[
        pl.BlockSpec((BF // 128, 128), lambda t: (0, 0)),
        pl.BlockSpec((VOCAB, NE), lambda t: (0, 0)),
        pl.BlockSpec((1, 1, NE), lambda t: (t, 0, 0)),
        pl.BlockSpec((VOCAB, NE), lambda t: (0, 0)),
        pl.BlockSpec((1, VOCAB), lambda t: (0, 0)),
    ],
    out_specs=[
        pl.BlockSpec((VOCAB, VOCAB), lambda t: (t, 0)),
        pl.BlockSpec((BF // 128, 128), lambda t: (0, 0)),
    ],
    out_shape=[
        jax.ShapeDtypeStruct((T * VOCAB, VOCAB), jnp.float32),
        jax.ShapeDtypeStruct((BF // 128, 128), jnp.int32),
    ],
)


def _gather_body(comb_hbm, fidx_hbm, out_hbm, idx_v, buf0, buf1, sg0, sg1, ss0, ss1):
    wid = lax.axis_index("s") * NC + lax.axis_index("c")
    base = wid * BPW
    pltpu.sync_copy(fidx_hbm.at[pl.ds(base, BPW)], idx_v)

    bufs = (buf0, buf1)
    sgs = (sg0, sg1)
    sss = (ss0, ss1)
    gathers = [None] * NCHUNK
    scatters = [None] * NCHUNK

    def start_gather(k):
        gathers[k] = pltpu.make_async_copy(
            comb_hbm.at[idx_v.at[pl.ds(k * CH, CH)]], bufs[k % 2], sgs[k % 2]
        )
        gathers[k].start()

    start_gather(0)
    for k in range(NCHUNK):
        b = k % 2
        gathers[k].wait()
        if k + 1 < NCHUNK:
            if k >= 1:
                scatters[k - 1].wait()
            start_gather(k + 1)
        scatters[k] = pltpu.make_async_copy(
            bufs[b], out_hbm.at[pl.ds(base + k * CH, CH)], sss[b]
        )
        scatters[k].start()
    scatters[NCHUNK - 2].wait()
    scatters[NCHUNK - 1].wait()


@functools.cache
def _make_gather():
    return pl.kernel(
        _gather_body,
        out_type=jax.ShapeDtypeStruct((BF, VOCAB), jnp.float32),
        mesh=plsc.VectorSubcoreMesh(
            core_axis_name="c", subcore_axis_name="s", num_cores=NC, num_subcores=NS
        ),
        scratch_types=[
            pltpu.VMEM((BPW,), jnp.int32),
            pltpu.VMEM((CH, VOCAB), jnp.float32),
            pltpu.VMEM((CH, VOCAB), jnp.float32),
            pltpu.SemaphoreType.DMA,
            pltpu.SemaphoreType.DMA,
            pltpu.SemaphoreType.DMA,
            pltpu.SemaphoreType.DMA,
        ],
        compiler_params=pltpu.CompilerParams(use_tc_tiling_on_sc=False),
    )


def kernel(idx, tok_table, pos_table, W, b):
    idx2 = idx.astype(jnp.int32).reshape(BF // 128, 128)
    comb, fidx = _tables(
        idx2,
        tok_table,
        pos_table.reshape(T, 1, NE),
        W,
        b.reshape(1, VOCAB),
    )
    out = _make_gather()(comb, fidx.reshape(BF))
    return out.reshape(BATCH, T, VOCAB)


# speedup vs baseline: 1.2839x; 1.2839x over previous
import functools

import jax
import jax.numpy as jnp
from jax import lax
from jax.experimental import pallas as pl
from jax.experimental.pallas import tpu as pltpu
from jax.experimental.pallas import tpu_sc as plsc

VOCAB = 1000
VOCAB_P = 1024
NE = 32
T = 8
BATCH = 4096
BF = BATCH * T

NC = 2
NS = 16
NW = NC * NS
BPW = BF // NW
CH = 32
NCHUNK = BPW // CH


def _tables_body(idx_ref, tok_ref, pos_ref, w_ref, b_ref, comb_ref, fidx_ref):
    t = pl.program_id(0)

    @pl.when(t == 0)
    def _():
        pos_of_lane = lax.broadcasted_iota(jnp.int32, (BF // 128, 128), 1) % T
        fidx_ref[...] = idx_ref[...] + pos_of_lane * VOCAB

    x = tok_ref[...] + pos_ref[0]
    comb_ref[...] = (
        lax.dot_general(
            x,
            w_ref[...],
            (((1,), (1,)), ((), ())),
            precision=lax.Precision.HIGHEST,
            preferred_element_type=jnp.float32,
        )
        + b_ref[...]
    )


_tables = pl.pallas_call(
    _tables_body,
    grid=(T,),
    in_specs=[
        pl.BlockSpec((BF // 128, 128), lambda t: (0, 0)),
        pl.BlockSpec((VOCAB, NE), lambda t: (0, 0)),
        pl.BlockSpec((1, 1, NE), lambda t: (t, 0, 0)),
        pl.BlockSpec((VOCAB_P, NE), lambda t: (0, 0)),
        pl.BlockSpec((1, VOCAB_P), lambda t: (0, 0)),
    ],
    out_specs=[
        pl.BlockSpec((VOCAB, VOCAB_P), lambda t: (t, 0)),
        pl.BlockSpec((BF // 128, 128), lambda t: (0, 0)),
    ],
    out_shape=[
        jax.ShapeDtypeStruct((T * VOCAB, VOCAB_P), jnp.float32),
        jax.ShapeDtypeStruct((BF // 128, 128), jnp.int32),
    ],
)


def _gather_body(comb_hbm, fidx_hbm, out_hbm, idx_v, buf0, buf1, sg0, sg1, ss0, ss1):
    wid = lax.axis_index("s") * NC + lax.axis_index("c")
    base = wid * BPW
    pltpu.sync_copy(fidx_hbm.at[pl.ds(base, BPW)], idx_v)

    bufs = (buf0, buf1)
    sgs = (sg0, sg1)
    sss = (ss0, ss1)
    gathers = [None] * NCHUNK
    scatters = [None] * NCHUNK

    def start_gather(k):
        gathers[k] = pltpu.make_async_copy(
            comb_hbm.at[idx_v.at[pl.ds(k * CH, CH)]], bufs[k % 2], sgs[k % 2]
        )
        gathers[k].start()

    start_gather(0)
    for k in range(NCHUNK):
        b = k % 2
        gathers[k].wait()
        if k + 1 < NCHUNK:
            if k >= 1:
                scatters[k - 1].wait()
            start_gather(k + 1)
        scatters[k] = pltpu.make_async_copy(
            bufs[b],
            out_hbm.at[pl.ds(base + k * CH, CH)],
            sss[b],
        )
        scatters[k].start()
    scatters[NCHUNK - 2].wait()
    scatters[NCHUNK - 1].wait()


@functools.cache
def _make_gather():
    return pl.kernel(
        _gather_body,
        out_type=jax.ShapeDtypeStruct((BF, VOCAB_P), jnp.float32),
        mesh=plsc.VectorSubcoreMesh(
            core_axis_name="c", subcore_axis_name="s", num_cores=NC, num_subcores=NS
        ),
        scratch_types=[
            pltpu.VMEM((BPW,), jnp.int32),
            pltpu.VMEM((CH, VOCAB_P), jnp.float32),
            pltpu.VMEM((CH, VOCAB_P), jnp.float32),
            pltpu.SemaphoreType.DMA,
            pltpu.SemaphoreType.DMA,
            pltpu.SemaphoreType.DMA,
            pltpu.SemaphoreType.DMA,
        ],
    )


def kernel(idx, tok_table, pos_table, W, b):
    idx2 = idx.astype(jnp.int32).reshape(BF // 128, 128)
    w_p = jnp.pad(W, ((0, VOCAB_P - VOCAB), (0, 0)))
    b_p = jnp.pad(b, (0, VOCAB_P - VOCAB)).reshape(1, VOCAB_P)
    comb, fidx = _tables(
        idx2,
        tok_table,
        pos_table.reshape(T, 1, NE),
        w_p,
        b_p,
    )
    out = _make_gather()(comb, fidx.reshape(BF))
    return out[:, :VOCAB].reshape(BATCH, T, VOCAB)


# speedup vs baseline: 3.4770x; 2.7081x over previous
import functools

import jax
import jax.numpy as jnp
from jax import lax
from jax.experimental import pallas as pl
from jax.experimental.pallas import tpu as pltpu
from jax.experimental.pallas import tpu_sc as plsc

VOCAB = 1000
NE = 32
T = 8
BATCH = 4096
BF = BATCH * T

NC = 2
NS = 16
NW = NC * NS
BPW = BATCH // NW
L = 16


def _emb_body(tok_hbm, idx_hbm, xt_hbm, tok_v, idx_v, xtl_v, lane16):
    wid = lax.axis_index("s") * NC + lax.axis_index("c")
    base = wid * (BPW * T)
    pltpu.sync_copy(tok_hbm, tok_v)
    pltpu.sync_copy(idx_hbm.at[pl.ds(base, BPW * T)], idx_v)
    lane = lane16[...]
    for t in range(T):
        for c in range(BPW // L):
            tok_idx = plsc.load_gather(idx_v, [lane * T + (c * L * T + t)])
            ti = tok_idx * NE
            for d in range(NE):
                xtl_v[t, d, pl.ds(c * L, L)] = plsc.load_gather(tok_v, [ti + d])
    pltpu.sync_copy(xtl_v, xt_hbm.at[:, :, pl.ds(wid * BPW, BPW)])


@functools.cache
def _make_emb():
    def body(tok_hbm, idx_hbm, xt_hbm, tok_v, idx_v, xtl_v):
        def inner(lane_ref):
            lane_ref[...] = lax.iota(jnp.int32, L)
            _emb_body(tok_hbm, idx_hbm, xt_hbm, tok_v, idx_v, xtl_v, lane_ref)

        pl.run_scoped(inner, pltpu.VMEM((L,), jnp.int32))

    return pl.kernel(
        body,
        out_type=jax.ShapeDtypeStruct((T, NE, BATCH), jnp.float32),
        mesh=plsc.VectorSubcoreMesh(
            core_axis_name="c", subcore_axis_name="s", num_cores=NC, num_subcores=NS
        ),
        scratch_types=[
            pltpu.VMEM((VOCAB * NE,), jnp.float32),
            pltpu.VMEM((BPW * T,), jnp.int32),
            pltpu.VMEM((T, NE, BPW), jnp.float32),
        ],
        compiler_params=pltpu.CompilerParams(needs_layout_passes=False),
    )


BN = 2048


def _dot_bf16x2(wh, wl, x, dims):
    xh = x.astype(jnp.bfloat16)
    xl = (x - xh.astype(jnp.float32)).astype(jnp.bfloat16)
    kw = dict(precision=lax.Precision.DEFAULT, preferred_element_type=jnp.float32)
    return (
        lax.dot_general(wh, xh, dims, **kw)
        + lax.dot_general(wh, xl, dims, **kw)
        + lax.dot_general(wl, xh, dims, **kw)
    )


def _head_body(xt_ref, w_ref, pos_ref, b_ref, out_ref):
    w = w_ref[...]
    wh = w.astype(jnp.bfloat16)
    wl = (w - wh.astype(jnp.float32)).astype(jnp.bfloat16)
    acc = _dot_bf16x2(wh, wl, xt_ref[0], (((1,), (0,)), ((), ())))
    pl_col = lax.dot_general(
        w,
        pos_ref[0],
        (((1,), (1,)), ((), ())),
        precision=lax.Precision.HIGHEST,
        preferred_element_type=jnp.float32,
    )
    out_ref[0] = acc + pl_col + b_ref[...]


_head = pl.pallas_call(
    _head_body,
    grid=(T, BATCH // BN),
    in_specs=[
        pl.BlockSpec((1, NE, BN), lambda t, nb: (t, 0, nb)),
        pl.BlockSpec((VOCAB, NE), lambda t, nb: (0, 0)),
        pl.BlockSpec((1, 1, NE), lambda t, nb: (t, 0, 0)),
        pl.BlockSpec((VOCAB, 1), lambda t, nb: (0, 0)),
    ],
    out_specs=pl.BlockSpec((1, VOCAB, BN), lambda t, nb: (t, 0, nb)),
    out_shape=jax.ShapeDtypeStruct((T, VOCAB, BATCH), jnp.float32),
)


def kernel(idx, tok_table, pos_table, W, b):
    idx_flat = idx.astype(jnp.int32).reshape(BF)
    tok_flat = tok_table.reshape(VOCAB * NE)
    xt = _make_emb()(tok_flat, idx_flat)
    ot = _head(xt, W, pos_table.reshape(T, 1, NE), b.reshape(VOCAB, 1))
    return jnp.transpose(ot, (2, 0, 1))


# speedup vs baseline: 3.8288x; 1.1012x over previous
import functools

import jax
import jax.numpy as jnp
from jax import lax
from jax.experimental import pallas as pl
from jax.experimental.pallas import tpu as pltpu
from jax.experimental.pallas import tpu_sc as plsc

VOCAB = 1000
NE = 32
T = 8
BATCH = 4096
BF = BATCH * T

NC = 2
NS = 16
NW = NC * NS
BPW = BATCH // NW
L = 16


ROWS_PW = (T * NE) // NW


def _emb_body(tok_hbm, idx_hbm, xt_hbm, tok_v, idx_v, xtl_v, lane16):
    wid = lax.axis_index("s") * NC + lax.axis_index("c")
    t = (wid * ROWS_PW) // NE
    d0 = (wid * ROWS_PW) % NE
    pltpu.sync_copy(tok_hbm, tok_v)
    pltpu.sync_copy(idx_hbm, idx_v)
    lane = lane16[...]
    for c in range(BATCH // L):
        tok_idx = plsc.load_gather(idx_v, [lane * T + (c * L * T + t)])
        ti = tok_idx * NE + d0
        for d in range(ROWS_PW):
            xtl_v[d, pl.ds(c * L, L)] = plsc.load_gather(tok_v, [ti + d])
    pltpu.sync_copy(xtl_v, xt_hbm.at[pl.ds(wid * ROWS_PW, ROWS_PW)])


@functools.cache
def _make_emb():
    def body(tok_hbm, idx_hbm, xt_hbm, tok_v, idx_v, xtl_v):
        def inner(lane_ref):
            lane_ref[...] = lax.iota(jnp.int32, L)
            _emb_body(tok_hbm, idx_hbm, xt_hbm, tok_v, idx_v, xtl_v, lane_ref)

        pl.run_scoped(inner, pltpu.VMEM((L,), jnp.int32))

    return pl.kernel(
        body,
        out_type=jax.ShapeDtypeStruct((T * NE, BATCH), jnp.float32),
        mesh=plsc.VectorSubcoreMesh(
            core_axis_name="c", subcore_axis_name="s", num_cores=NC, num_subcores=NS
        ),
        scratch_types=[
            pltpu.VMEM((VOCAB * NE,), jnp.float32),
            pltpu.VMEM((BF,), jnp.int32),
            pltpu.VMEM((ROWS_PW, BATCH), jnp.float32),
        ],
        compiler_params=pltpu.CompilerParams(needs_layout_passes=False),
    )


BN = 2048


def _dot_bf16x2(wh, wl, x, dims):
    xh = x.astype(jnp.bfloat16)
    xl = (x - xh.astype(jnp.float32)).astype(jnp.bfloat16)
    kw = dict(precision=lax.Precision.DEFAULT, preferred_element_type=jnp.float32)
    return (
        lax.dot_general(wh, xh, dims, **kw)
        + lax.dot_general(wh, xl, dims, **kw)
        + lax.dot_general(wl, xh, dims, **kw)
    )


def _head_body(xt_ref, w_ref, pos_ref, b_ref, out_ref):
    w = w_ref[...]
    acc = lax.dot_general(
        w.astype(jnp.bfloat16),
        xt_ref[0].astype(jnp.bfloat16),
        (((1,), (0,)), ((), ())),
        preferred_element_type=jnp.float32,
    )
    pl_col = lax.dot_general(
        w,
        pos_ref[0],
        (((1,), (1,)), ((), ())),
        precision=lax.Precision.HIGHEST,
        preferred_element_type=jnp.float32,
    )
    out_ref[0] = acc + pl_col + b_ref[...]


_head = pl.pallas_call(
    _head_body,
    grid=(T, BATCH // BN),
    in_specs=[
        pl.BlockSpec((1, NE, BN), lambda t, nb: (t, 0, nb)),
        pl.BlockSpec((VOCAB, NE), lambda t, nb: (0, 0)),
        pl.BlockSpec((1, 1, NE), lambda t, nb: (t, 0, 0)),
        pl.BlockSpec((VOCAB, 1), lambda t, nb: (0, 0)),
    ],
    out_specs=pl.BlockSpec((1, VOCAB, BN), lambda t, nb: (t, 0, nb)),
    out_shape=jax.ShapeDtypeStruct((T, VOCAB, BATCH), jnp.float32),
)


def kernel(idx, tok_table, pos_table, W, b):
    idx_flat = idx.astype(jnp.int32).reshape(BF)
    tok_flat = tok_table.reshape(VOCAB * NE)
    xt = _make_emb()(tok_flat, idx_flat).reshape(T, NE, BATCH)
    ot = _head(xt, W, pos_table.reshape(T, 1, NE), b.reshape(VOCAB, 1))
    return jnp.transpose(ot, (2, 0, 1))


# speedup vs baseline: 4.7935x; 1.2520x over previous
import functools

import jax
import jax.numpy as jnp
from jax import lax
from jax.experimental import pallas as pl
from jax.experimental.pallas import tpu as pltpu
from jax.experimental.pallas import tpu_sc as plsc

VOCAB = 1000
NE = 32
T = 8
BATCH = 4096
BF = BATCH * T

NC = 2
NS = 16
NW = NC * NS
BPW = BATCH // NW
L = 16


ROWS_PW = (T * NE) // NW


def _emb_body(tok_hbm, idx_hbm, xt_hbm, tok_v, idx_v, xtl_v):
    wid = lax.axis_index("s") * NC + lax.axis_index("c")
    t = (wid * ROWS_PW) // NE
    d0 = (wid * ROWS_PW) % NE
    pltpu.sync_copy(tok_hbm, tok_v)
    pltpu.sync_copy(idx_hbm.at[pl.ds(t * BATCH, BATCH)], idx_v)

    @plsc.parallel_loop(0, BATCH, step=L, unroll=8)
    def _(i):
        ti = idx_v[pl.ds(i, L)] * NE + d0
        for d in range(ROWS_PW):
            xtl_v[d, pl.ds(i, L)] = plsc.load_gather(tok_v, [ti + d])

    pltpu.sync_copy(xtl_v, xt_hbm.at[pl.ds(wid * ROWS_PW, ROWS_PW)])


@functools.cache
def _make_emb():
    return pl.kernel(
        _emb_body,
        out_type=jax.ShapeDtypeStruct((T * NE, BATCH), jnp.float32),
        mesh=plsc.VectorSubcoreMesh(
            core_axis_name="c", subcore_axis_name="s", num_cores=NC, num_subcores=NS
        ),
        scratch_types=[
            pltpu.VMEM((VOCAB * NE,), jnp.float32),
            pltpu.VMEM((BATCH,), jnp.int32),
            pltpu.VMEM((ROWS_PW, BATCH), jnp.float32),
        ],
        compiler_params=pltpu.CompilerParams(needs_layout_passes=False),
    )


BN = 2048


def _dot_bf16x2(wh, wl, x, dims):
    xh = x.astype(jnp.bfloat16)
    xl = (x - xh.astype(jnp.float32)).astype(jnp.bfloat16)
    kw = dict(precision=lax.Precision.DEFAULT, preferred_element_type=jnp.float32)
    return (
        lax.dot_general(wh, xh, dims, **kw)
        + lax.dot_general(wh, xl, dims, **kw)
        + lax.dot_general(wl, xh, dims, **kw)
    )


def _head_body(xt_ref, w_ref, pos_ref, b_ref, out_ref):
    w = w_ref[...]
    acc = lax.dot_general(
        w.astype(jnp.bfloat16),
        xt_ref[0].astype(jnp.bfloat16),
        (((1,), (0,)), ((), ())),
        preferred_element_type=jnp.float32,
    )
    pl_col = lax.dot_general(
        w,
        pos_ref[0],
        (((1,), (1,)), ((), ())),
        precision=lax.Precision.HIGHEST,
        preferred_element_type=jnp.float32,
    )
    out_ref[0] = acc + pl_col + b_ref[...]


_head = pl.pallas_call(
    _head_body,
    grid=(T, BATCH // BN),
    in_specs=[
        pl.BlockSpec((1, NE, BN), lambda t, nb: (t, 0, nb)),
        pl.BlockSpec((VOCAB, NE), lambda t, nb: (0, 0)),
        pl.BlockSpec((1, 1, NE), lambda t, nb: (t, 0, 0)),
        pl.BlockSpec((VOCAB, 1), lambda t, nb: (0, 0)),
    ],
    out_specs=pl.BlockSpec((1, VOCAB, BN), lambda t, nb: (t, 0, nb)),
    out_shape=jax.ShapeDtypeStruct((T, VOCAB, BATCH), jnp.float32),
)


def kernel(idx, tok_table, pos_table, W, b):
    idx_flat = idx.astype(jnp.int32).T.reshape(BF)
    tok_flat = tok_table.reshape(VOCAB * NE)
    xt = _make_emb()(tok_flat, idx_flat).reshape(T, NE, BATCH)
    ot = _head(xt, W, pos_table.reshape(T, 1, NE), b.reshape(VOCAB, 1))
    return jnp.transpose(ot, (2, 0, 1))


# speedup vs baseline: 5.7896x; 1.2078x over previous
import functools

import jax
import jax.numpy as jnp
from jax import lax
from jax.experimental import pallas as pl
from jax.experimental.pallas import tpu as pltpu
from jax.experimental.pallas import tpu_sc as plsc

VOCAB = 1000
NE = 32
T = 8
BATCH = 4096
BF = BATCH * T

NC = 2
NS = 16
NW = NC * NS
BPW = BATCH // NW
L = 16


ROWS_PW = (T * NE) // NW


def _emb_body(tok_hbm, idx_hbm, xt_hbm, tok_v, idx_v, xtl_v):
    wid = lax.axis_index("s") * NC + lax.axis_index("c")
    t = (wid * ROWS_PW) // NE
    d0 = (wid * ROWS_PW) % NE
    pltpu.sync_copy(tok_hbm, tok_v)
    pltpu.sync_copy(idx_hbm.at[pl.ds(t * BATCH, BATCH)], idx_v)

    @plsc.parallel_loop(0, BATCH, step=L, unroll=8)
    def _(i):
        ti = idx_v[pl.ds(i, L)] + d0 * VOCAB
        for d in range(ROWS_PW):
            xtl_v[d, pl.ds(i, L)] = plsc.load_gather(tok_v, [ti + d * VOCAB])

    pltpu.sync_copy(xtl_v, xt_hbm.at[pl.ds(wid * ROWS_PW, ROWS_PW)])


@functools.cache
def _make_emb():
    return pl.kernel(
        _emb_body,
        out_type=jax.ShapeDtypeStruct((T * NE, BATCH), jnp.float32),
        mesh=plsc.VectorSubcoreMesh(
            core_axis_name="c", subcore_axis_name="s", num_cores=NC, num_subcores=NS
        ),
        scratch_types=[
            pltpu.VMEM((VOCAB * NE,), jnp.float32),
            pltpu.VMEM((BATCH,), jnp.int32),
            pltpu.VMEM((ROWS_PW, BATCH), jnp.float32),
        ],
        compiler_params=pltpu.CompilerParams(needs_layout_passes=False),
    )


BN = 2048


def _dot_bf16x2(wh, wl, x, dims):
    xh = x.astype(jnp.bfloat16)
    xl = (x - xh.astype(jnp.float32)).astype(jnp.bfloat16)
    kw = dict(precision=lax.Precision.DEFAULT, preferred_element_type=jnp.float32)
    return (
        lax.dot_general(wh, xh, dims, **kw)
        + lax.dot_general(wh, xl, dims, **kw)
        + lax.dot_general(wl, xh, dims, **kw)
    )


def _head_body(xt_ref, w_ref, pos_ref, b_ref, out_ref):
    w = w_ref[...]
    acc = lax.dot_general(
        w.astype(jnp.bfloat16),
        xt_ref[0].astype(jnp.bfloat16),
        (((1,), (0,)), ((), ())),
        preferred_element_type=jnp.float32,
    )
    pl_col = lax.dot_general(
        w,
        pos_ref[0],
        (((1,), (1,)), ((), ())),
        precision=lax.Precision.HIGHEST,
        preferred_element_type=jnp.float32,
    )
    out_ref[0] = acc + pl_col + b_ref[...]


_head = pl.pallas_call(
    _head_body,
    grid=(T, BATCH // BN),
    in_specs=[
        pl.BlockSpec((1, NE, BN), lambda t, nb: (t, 0, nb)),
        pl.BlockSpec((VOCAB, NE), lambda t, nb: (0, 0)),
        pl.BlockSpec((1, 1, NE), lambda t, nb: (t, 0, 0)),
        pl.BlockSpec((VOCAB, 1), lambda t, nb: (0, 0)),
    ],
    out_specs=pl.BlockSpec((1, VOCAB, BN), lambda t, nb: (t, 0, nb)),
    out_shape=jax.ShapeDtypeStruct((T, VOCAB, BATCH), jnp.float32),
)


def kernel(idx, tok_table, pos_table, W, b):
    idx_flat = idx.astype(jnp.int32).T.reshape(BF)
    tok_flat = tok_table.T.reshape(VOCAB * NE)
    xt = _make_emb()(tok_flat, idx_flat).reshape(T, NE, BATCH)
    ot = _head(xt, W, pos_table.reshape(T, 1, NE), b.reshape(VOCAB, 1))
    return jnp.transpose(ot, (2, 0, 1))
